# baseline (speedup 1.0000x reference)
import functools

import jax
import jax.numpy as jnp
from jax import lax
from jax.experimental import pallas as pl
from jax.experimental.pallas import tpu as pltpu
from jax.experimental.pallas import tpu_sc as plsc

B, L, D = 4, 2048, 768
E, K, H = 8, 2, 16
NC, NS, LANES = 2, 16, 16
NW = NC * NS
DSLICES = NW // B
GROUPS = D * H // 128
GPW = 16
AW = GROUPS // GPW
NEG = -1e30


def _logits_kernel(x_ref, w_gate_ref, out_ref):
    xb = x_ref[0]
    gx = jnp.sum(xb, axis=0, keepdims=True) * (1.0 / L)
    lg = jnp.dot(gx, w_gate_ref[...],
                 preferred_element_type=jnp.float32)
    out_ref[0] = jnp.concatenate(
        [lg, jnp.full((1, LANES - E), NEG, jnp.float32)], axis=1)


def _top2(v, lane):
    m1 = jnp.max(v)
    i1 = jnp.min(jnp.where(v == m1, lane, LANES))
    i1v = jnp.full((LANES,), i1, jnp.int32)
    v2 = jnp.where(lane == i1v, NEG, v)
    m2 = jnp.max(v2)
    i2 = jnp.min(jnp.where(v2 == m2, lane, LANES))
    i2v = jnp.full((LANES,), i2, jnp.int32)
    t = jnp.exp(jnp.full((LANES,), m2 - m1, jnp.float32))
    g1v = 1.0 / (1.0 + t)
    g2v = t * g1v
    return g1v, g2v, i1v, i2v


def _route_kernel(logits_hbm, ew_hbm, eb_hbm, shw_hbm, shb_hbm,
                  m_hbm, bias_hbm, loss_hbm,
                  idx_v, rows1_v, rows2_v, shv_v, ebv_v, shbv_v,
                  loss_v, bias4_v, lgall_v, sem):
    wid = lax.axis_index("c") * NS + lax.axis_index("s")
    b = wid // DSLICES
    sub = wid % DSLICES
    g0 = sub * GPW
    lane = lax.iota(jnp.int32, LANES)

    pltpu.sync_copy(logits_hbm, lgall_v)

    @pl.when(sub < AW)
    def _():
        bv = jnp.full((LANES,), b, jnp.int32)
        v = jnp.zeros((LANES,), jnp.float32)
        for bb in range(B):
            v = v + jnp.where(bv == bb, lgall_v[bb], 0.0)
        g1v, g2v, i1v, i2v = _top2(v, lane)

        idx_v[...] = i1v * GROUPS + g0 + lane
        pltpu.async_copy(ew_hbm.at[idx_v], rows1_v, sem).wait()
        idx_v[...] = i2v * GROUPS + g0 + lane
        pltpu.async_copy(ew_hbm.at[idx_v], rows2_v, sem).wait()
        pltpu.sync_copy(shw_hbm.at[pl.ds(g0, GPW)], shv_v)

        for i in range(GPW):
            for s in range(8):
                sl = pl.ds(s * LANES, LANES)
                rows1_v[i, sl] = (g1v * rows1_v[i, sl]
                                  + g2v * rows2_v[i, sl] + shv_v[i, sl])
        pltpu.sync_copy(rows1_v, m_hbm.at[pl.ds(b * GROUPS + g0, GPW)])

    @pl.when(wid == 0)
    def _():
        pltpu.sync_copy(eb_hbm, ebv_v)
        pltpu.sync_copy(shb_hbm, shbv_v)
        imp = jnp.zeros((LANES,), jnp.float32)
        ld = jnp.zeros((LANES,), jnp.float32)
        for bb in range(B):
            g1b, g2b, i1b, i2b = _top2(lgall_v[bb], lane)
            grow = (jnp.where(lane == i1b, g1b, 0.0)
                    + jnp.where(lane == i2b, g2b, 0.0))
            imp = imp + grow
            ld = ld + jnp.where(grow > 0.0, 1.0, 0.0)
            bb_bias = shbv_v[0]
            for e in range(E):
                coef = (jnp.where(i1b == e, g1b, 0.0)
                        + jnp.where(i2b == e, g2b, 0.0))
                bb_bias = bb_bias + coef * ebv_v[e]
            bias4_v[bb] = bb_bias
        pltpu.sync_copy(bias4_v, bias_hbm)

        emask = lane < E
        eps = 1e-10

        def cv2(v):
            mean = jnp.full((LANES,), jnp.sum(v) * (1.0 / E), jnp.float32)
            diff = jnp.where(emask, v - mean, 0.0)
            var = jnp.full((LANES,), jnp.sum(diff * diff) * (1.0 / (E - 1)),
                           jnp.float32)
            return var / (mean * mean + eps)

        loss_v[0] = (cv2(imp) + cv2(ld)) * 1e-2
        pltpu.sync_copy(loss_v, loss_hbm)


_route = functools.partial(
    pl.kernel,
    out_type=[
        jax.ShapeDtypeStruct((B * GROUPS, 128), jnp.float32),
        jax.ShapeDtypeStruct((B, H), jnp.float32),
        jax.ShapeDtypeStruct((1, LANES), jnp.float32),
    ],
    mesh=plsc.VectorSubcoreMesh(core_axis_name="c", subcore_axis_name="s"),
    scratch_types=[
        pltpu.VMEM((LANES,), jnp.int32),
        pltpu.VMEM((GPW, 128), jnp.float32),
        pltpu.VMEM((GPW, 128), jnp.float32),
        pltpu.VMEM((GPW, 128), jnp.float32),
        pltpu.VMEM((E, H), jnp.float32),
        pltpu.VMEM((1, LANES), jnp.float32),
        pltpu.VMEM((1, LANES), jnp.float32),
        pltpu.VMEM((B, LANES), jnp.float32),
        pltpu.VMEM((B, LANES), jnp.float32),
        pltpu.SemaphoreType.DMA,
    ],
    compiler_params=pltpu.CompilerParams(needs_layout_passes=False),
)(_route_kernel)


def _matmul_kernel(x_ref, m_ref, bias_ref, out_ref):
    yb = lax.dot_general(
        x_ref[0].astype(jnp.bfloat16), m_ref[0].astype(jnp.bfloat16),
        (((1,), (0,)), ((), ())),
        preferred_element_type=jnp.float32)
    out_ref[0] = yb + bias_ref[0]


@jax.jit
def kernel(x, w_gate, expert_w, expert_b, shared_w, shared_b):
    logits = pl.pallas_call(
        _logits_kernel,
        grid=(B,),
        in_specs=[
            pl.BlockSpec((1, L, D), lambda b: (b, 0, 0)),
            pl.BlockSpec((D, E), lambda b: (0, 0)),
        ],
        out_specs=pl.BlockSpec((1, 1, LANES), lambda b: (b, 0, 0)),
        out_shape=jax.ShapeDtypeStruct((B, 1, LANES), jnp.float32),
    )(x, w_gate)

    m_flat, bias, loss = _route(
        logits.reshape(B, LANES), expert_w.reshape(E * GROUPS, 128),
        expert_b, shared_w.reshape(GROUPS, 128), shared_b.reshape(1, H))

    out = pl.pallas_call(
        _matmul_kernel,
        grid=(B,),
        in_specs=[
            pl.BlockSpec((1, L, D), lambda b: (b, 0, 0)),
            pl.BlockSpec((1, D, H), lambda b: (b, 0, 0)),
            pl.BlockSpec((1, 1, H), lambda b: (b, 0, 0)),
        ],
        out_specs=pl.BlockSpec((1, L, H), lambda b: (b, 0, 0)),
        out_shape=jax.ShapeDtypeStruct((B, L, H), jnp.float32),
    )(x, m_flat.reshape(B, D, H), bias.reshape(B, 1, H))
    return out, loss[0, 0]


# speedup vs baseline: 1.3979x; 1.3979x over previous
import functools

import jax
import jax.numpy as jnp
from jax import lax
from jax.experimental import pallas as pl
from jax.experimental.pallas import tpu as pltpu
from jax.experimental.pallas import tpu_sc as plsc

B, L, D = 4, 2048, 768
E, K, H = 8, 2, 16
NS, LANES = 16, 16


def _moe_kernel(x_ref, w_gate_ref, expert_w_ref, expert_b_ref,
                shared_w_ref, shared_b_ref, out_ref, gates_ref):
    xb = x_ref[0]

    gx = jnp.sum(xb, axis=0, keepdims=True) * (1.0 / L)
    logits = jnp.dot(gx, w_gate_ref[...],
                     preferred_element_type=jnp.float32)

    lane = jax.lax.broadcasted_iota(jnp.int32, (1, E), 1)
    m1 = jnp.max(logits)
    i1 = jnp.min(jnp.where(logits == m1, lane, E))
    mask1 = lane == i1
    l2 = jnp.where(mask1, -jnp.inf, logits)
    m2 = jnp.max(l2)
    i2 = jnp.min(jnp.where(l2 == m2, lane, E))
    mask2 = lane == i2
    t = jnp.exp(m2 - m1)
    g1 = 1.0 / (1.0 + t)
    g2 = t / (1.0 + t)
    gates_row = jnp.where(mask1, g1, 0.0) + jnp.where(mask2, g2, 0.0)
    gates_ref[0] = jnp.concatenate(
        [gates_row, jnp.zeros((1, LANES - E), jnp.float32)], axis=1)

    m_w = shared_w_ref[...]
    bias = shared_b_ref[...]
    for e in range(E):
        ge = jnp.sum(jnp.where(lane == e, gates_row, 0.0))
        m_w = m_w + ge * expert_w_ref[e]
        bias = bias + ge * expert_b_ref[e][None, :]

    yb = jax.lax.dot_general(
        xb.astype(jnp.bfloat16), m_w.astype(jnp.bfloat16),
        (((1,), (0,)), ((), ())),
        preferred_element_type=jnp.float32)
    out_ref[0] = yb + bias


def _loss_kernel(gates_hbm, loss_hbm, gall_v, loss_v):
    wid = lax.axis_index("c") * NS + lax.axis_index("s")

    @pl.when(wid == 0)
    def _():
        pltpu.sync_copy(gates_hbm, gall_v)
        lane = lax.iota(jnp.int32, LANES)
        imp = jnp.zeros((LANES,), jnp.float32)
        ld = jnp.zeros((LANES,), jnp.float32)
        for bb in range(B):
            grow = gall_v[bb]
            imp = imp + grow
            ld = ld + jnp.where(grow > 0.0, 1.0, 0.0)
        emask = lane < E
        eps = 1e-10

        def cv2(v):
            mean = jnp.full((LANES,), jnp.sum(v) * (1.0 / E), jnp.float32)
            diff = jnp.where(emask, v - mean, 0.0)
            var = jnp.full((LANES,), jnp.sum(diff * diff) * (1.0 / (E - 1)),
                           jnp.float32)
            return var / (mean * mean + eps)

        loss_v[0] = (cv2(imp) + cv2(ld)) * 1e-2
        pltpu.sync_copy(loss_v, loss_hbm)


_loss = functools.partial(
    pl.kernel,
    out_type=jax.ShapeDtypeStruct((1, LANES), jnp.float32),
    mesh=plsc.VectorSubcoreMesh(core_axis_name="c", subcore_axis_name="s"),
    scratch_types=[
        pltpu.VMEM((B, LANES), jnp.float32),
        pltpu.VMEM((1, LANES), jnp.float32),
    ],
    compiler_params=pltpu.CompilerParams(needs_layout_passes=False),
)(_loss_kernel)


@jax.jit
def kernel(x, w_gate, expert_w, expert_b, shared_w, shared_b):
    out, gates = pl.pallas_call(
        _moe_kernel,
        grid=(B,),
        in_specs=[
            pl.BlockSpec((1, L, D), lambda b: (b, 0, 0)),
            pl.BlockSpec((D, E), lambda b: (0, 0)),
            pl.BlockSpec((E, D, H), lambda b: (0, 0, 0)),
            pl.BlockSpec((E, H), lambda b: (0, 0)),
            pl.BlockSpec((D, H), lambda b: (0, 0)),
            pl.BlockSpec((1, H), lambda b: (0, 0)),
        ],
        out_specs=[
            pl.BlockSpec((1, L, H), lambda b: (b, 0, 0)),
            pl.BlockSpec((1, 1, LANES), lambda b: (b, 0, 0)),
        ],
        out_shape=[
            jax.ShapeDtypeStruct((B, L, H), jnp.float32),
            jax.ShapeDtypeStruct((B, 1, LANES), jnp.float32),
        ],
    )(x, w_gate, expert_w, expert_b, shared_w, shared_b.reshape(1, H))

    loss = _loss(gates.reshape(B, LANES))
    return out, loss[0, 0]


# speedup vs baseline: 2.1728x; 1.5543x over previous
import functools

import jax
import jax.numpy as jnp
from jax.experimental import pallas as pl
from jax.experimental.pallas import tpu as pltpu

B, L, D = 4, 2048, 768
E, K, H = 8, 2, 16
NSPLIT = 2
LS = L // NSPLIT


def _moe_kernel(*refs):
    x_refs = refs[:NSPLIT]
    (w_gate_ref, expert_w_ref, expert_b_ref, shared_w_ref, shared_b_ref,
     out_ref, loss_ref, gates_acc) = refs[NSPLIT:]
    b = pl.program_id(0)
    nb = pl.num_programs(0)

    gx = x_refs[0][0].sum(axis=0, keepdims=True)
    for r in x_refs[1:]:
        gx = gx + r[0].sum(axis=0, keepdims=True)
    gx = gx * (1.0 / L)
    logits = jnp.dot(gx, w_gate_ref[...],
                     preferred_element_type=jnp.float32)

    lane = jax.lax.broadcasted_iota(jnp.int32, (1, E), 1)
    m1 = jnp.max(logits)
    i1 = jnp.min(jnp.where(logits == m1, lane, E))
    mask1 = lane == i1
    l2 = jnp.where(mask1, -jnp.inf, logits)
    m2 = jnp.max(l2)
    i2 = jnp.min(jnp.where(l2 == m2, lane, E))
    mask2 = lane == i2
    t = jnp.exp(m2 - m1)
    g1 = 1.0 / (1.0 + t)
    g2 = t / (1.0 + t)
    gates_row = jnp.where(mask1, g1, 0.0) + jnp.where(mask2, g2, 0.0)

    m_w = shared_w_ref[...]
    bias = shared_b_ref[...]
    for e in range(E):
        ge = jnp.sum(jnp.where(lane == e, gates_row, 0.0))
        m_w = m_w + ge * expert_w_ref[e]
        bias = bias + ge * expert_b_ref[e][None, :]
    for i, r in enumerate(x_refs):
        yb = jax.lax.dot_general(
            r[0], m_w,
            (((1,), (0,)), ((), ())),
            preferred_element_type=jnp.float32)
        out_ref[0, i * LS:(i + 1) * LS, :] = yb + bias

    row = jax.lax.broadcasted_iota(jnp.int32, (B, E), 0)

    @pl.when(b == 0)
    def _():
        gates_acc[...] = jnp.zeros((B, E), jnp.float32)

    gates_acc[...] = jnp.where(row == b, gates_row, gates_acc[...])

    @pl.when(b == nb - 1)
    def _():
        gates_all = gates_acc[...]
        eps = 1e-10

        def cv2(v):
            mean = jnp.sum(v) * (1.0 / E)
            var = jnp.sum((v - mean) ** 2) * (1.0 / (E - 1))
            return var / (mean * mean + eps)

        importance = jnp.sum(gates_all, axis=0, keepdims=True)
        load = jnp.sum((gates_all > 0).astype(jnp.float32), axis=0,
                       keepdims=True)
        loss_ref[...] = jnp.full((1, 1), (cv2(importance) + cv2(load)) * 1e-2,
                                 jnp.float32)


@functools.partial(jax.jit, static_argnames=("interpret",))
def kernel(x, w_gate, expert_w, expert_b, shared_w, shared_b,
           interpret=False):
    x_specs = [
        pl.BlockSpec((1, LS, D), functools.partial(lambda i, b: (b, i, 0), i))
        for i in range(NSPLIT)
    ]
    out, loss = pl.pallas_call(
        _moe_kernel,
        grid=(B,),
        in_specs=x_specs + [
            pl.BlockSpec((D, E), lambda b: (0, 0)),
            pl.BlockSpec((E, D, H), lambda b: (0, 0, 0)),
            pl.BlockSpec((E, H), lambda b: (0, 0)),
            pl.BlockSpec((D, H), lambda b: (0, 0)),
            pl.BlockSpec((1, H), lambda b: (0, 0)),
        ],
        out_specs=[
            pl.BlockSpec((1, L, H), lambda b: (b, 0, 0)),
            pl.BlockSpec((1, 1), lambda b: (0, 0)),
        ],
        out_shape=[
            jax.ShapeDtypeStruct((B, L, H), jnp.float32),
            jax.ShapeDtypeStruct((1, 1), jnp.float32),
        ],
        scratch_shapes=[pltpu.VMEM((B, E), jnp.float32)],
        interpret=interpret,
    )(*([x] * NSPLIT), w_gate, expert_w, expert_b, shared_w,
      shared_b.reshape(1, H))
    return out, loss[0, 0]
